# Initial kernel scaffold; baseline (speedup 1.0000x reference)
import functools

import jax
import jax.numpy as jnp
from jax import lax
from jax.experimental import pallas as pl
from jax.experimental.pallas import tpu as pltpu
from jax.experimental.pallas import tpu_sc as plsc

DIM = 128
L_SIZE = 10000
C_SIZE = 5000
N_EDGES = 320000
N_ITER = 4

L_PAD = 10240
C_PAD = 5120
NC = 2
NS = 16
NW = NC * NS
K = 128
E_PAD = 327680
EPW = E_PAD // NW
CHUNKS = EPW // K

_f32 = jnp.float32


def _dot(x, w):
    return lax.dot_general(x, w, (((1,), (1,)), ((), ())),
                           preferred_element_type=_f32)


def _pre_l_body(x_ref, w1_ref, b1_ref, w2_ref, b2_ref,
                v1_ref, vb1_ref, v2_ref, vb2_ref, msg_ref, l2l_ref):
    x = x_ref[...]
    h = jnp.maximum(_dot(x, w1_ref[...]) + b1_ref[...], 0.0)
    msg_ref[...] = _dot(h, w2_ref[...]) + b2_ref[...]
    xs = x.reshape(-1, 2, DIM)
    xsw = jnp.concatenate([xs[:, 1:2, :], xs[:, 0:1, :]], axis=1)
    xsw = xsw.reshape(x.shape)
    h2 = jnp.maximum(_dot(xsw, v1_ref[...]) + vb1_ref[...], 0.0)
    l2l_ref[...] = _dot(h2, v2_ref[...]) + vb2_ref[...]


def _pre_c_body(x_ref, w1_ref, b1_ref, w2_ref, b2_ref, msg_ref):
    x = x_ref[...]
    h = jnp.maximum(_dot(x, w1_ref[...]) + b1_ref[...], 0.0)
    msg_ref[...] = _dot(h, w2_ref[...]) + b2_ref[...]


def _gru_gates(gi, gh, h):
    ir, iz, inn = gi[:, :DIM], gi[:, DIM:2 * DIM], gi[:, 2 * DIM:]
    hr, hz, hn = gh[:, :DIM], gh[:, DIM:2 * DIM], gh[:, 2 * DIM:]
    r = jax.nn.sigmoid(ir + hr)
    z = jax.nn.sigmoid(iz + hz)
    n = jnp.tanh(inn + r * hn)
    return (1.0 - z) * n + z * h


def _gru_c_body(agg_ref, h_ref, wih_ref, whh_ref, bih_ref, bhh_ref, out_ref):
    x = agg_ref[0] + agg_ref[1]
    h = h_ref[...]
    gi = _dot(x, wih_ref[...]) + bih_ref[...]
    gh = _dot(h, whh_ref[...]) + bhh_ref[...]
    out_ref[...] = _gru_gates(gi, gh, h)


def _gru_l_body(agg_ref, l2l_ref, h_ref, wih_ref, whh_ref, bih_ref, bhh_ref,
                out_ref):
    x = jnp.concatenate([agg_ref[0] + agg_ref[1], l2l_ref[...]], axis=1)
    h = h_ref[...]
    gi = _dot(x, wih_ref[...]) + bih_ref[...]
    gh = _dot(h, whh_ref[...]) + bhh_ref[...]
    out_ref[...] = _gru_gates(gi, gh, h)


def _full(shape):
    return pl.BlockSpec(shape, lambda i: tuple(0 for _ in shape))


def _rows(block):
    return pl.BlockSpec((block, DIM), lambda i: (i, 0))


def _pre_l(x, w1, b1, w2, b2, v1, vb1, v2, vb2):
    blk = 512
    grid = (L_PAD // blk,)
    return pl.pallas_call(
        _pre_l_body,
        grid=grid,
        in_specs=[_rows(blk), _full((DIM, DIM)), _full((1, DIM)),
                  _full((DIM, DIM)), _full((1, DIM)), _full((DIM, DIM)),
                  _full((1, DIM)), _full((DIM, DIM)), _full((1, DIM))],
        out_specs=[_rows(blk), _rows(blk)],
        out_shape=[jax.ShapeDtypeStruct((L_PAD, DIM), _f32),
                   jax.ShapeDtypeStruct((L_PAD, DIM), _f32)],
    )(x, w1, b1.reshape(1, DIM), w2, b2.reshape(1, DIM),
      v1, vb1.reshape(1, DIM), v2, vb2.reshape(1, DIM))


def _pre_c(x, w1, b1, w2, b2):
    blk = 512
    grid = (C_PAD // blk,)
    return pl.pallas_call(
        _pre_c_body,
        grid=grid,
        in_specs=[_rows(blk), _full((DIM, DIM)), _full((1, DIM)),
                  _full((DIM, DIM)), _full((1, DIM))],
        out_specs=_rows(blk),
        out_shape=jax.ShapeDtypeStruct((C_PAD, DIM), _f32),
    )(x, w1, b1.reshape(1, DIM), w2, b2.reshape(1, DIM))


def _agg_spec(blk):
    return pl.BlockSpec((2, blk, DIM), lambda i: (0, i, 0))


def _gru_c(agg, h, wih, whh, bih, bhh):
    blk = 512
    grid = (C_PAD // blk,)
    return pl.pallas_call(
        _gru_c_body,
        grid=grid,
        in_specs=[_agg_spec(blk), _rows(blk), _full((3 * DIM, DIM)),
                  _full((3 * DIM, DIM)), _full((1, 3 * DIM)),
                  _full((1, 3 * DIM))],
        out_specs=_rows(blk),
        out_shape=jax.ShapeDtypeStruct((C_PAD, DIM), _f32),
    )(agg, h, wih, whh, bih.reshape(1, 3 * DIM), bhh.reshape(1, 3 * DIM))


def _gru_l(agg, l2l, h, wih, whh, bih, bhh):
    blk = 512
    grid = (L_PAD // blk,)
    return pl.pallas_call(
        _gru_l_body,
        grid=grid,
        in_specs=[_agg_spec(blk), _rows(blk), _rows(blk),
                  _full((3 * DIM, 2 * DIM)), _full((3 * DIM, DIM)),
                  _full((1, 3 * DIM)), _full((1, 3 * DIM))],
        out_specs=_rows(blk),
        out_shape=jax.ShapeDtypeStruct((L_PAD, DIM), _f32),
    )(agg, l2l, h, wih, whh, bih.reshape(1, 3 * DIM), bhh.reshape(1, 3 * DIM))


_ZR = 64
_C_STRIPE = C_PAD // NS
_L_STRIPE = L_PAD // NS


def _sc_body(l_idx, c_idx, l_tab, c_tab, out_c, out_l,
             lidx_v, cidx_v, lrows_v, crows_v, zbuf_v, cacc, lacc,
             sem1, sem2):
    cid = lax.axis_index("c")
    sid = lax.axis_index("s")
    wid = cid * NS + sid

    zeros16 = jnp.zeros((16,), _f32)

    def _zrow(i, carry):
        for j in range(DIM // 16):
            zbuf_v[i, pl.ds(j * 16, 16)] = zeros16
        return carry

    lax.fori_loop(0, _ZR, _zrow, 0)

    def _zc(j, carry):
        base = pl.multiple_of(sid * _C_STRIPE + j * _ZR, _ZR)
        pltpu.sync_copy(zbuf_v, cacc.at[pl.ds(base, _ZR)])
        return carry

    lax.fori_loop(0, _C_STRIPE // _ZR, _zc, 0)

    def _zl(j, carry):
        base = pl.multiple_of(sid * _L_STRIPE + j * _ZR, _ZR)
        pltpu.sync_copy(zbuf_v, lacc.at[pl.ds(base, _ZR)])
        return carry

    lax.fori_loop(0, _L_STRIPE // _ZR, _zl, 0)
    plsc.subcore_barrier()

    ebase = wid * EPW

    def _chunk(j, carry):
        off = pl.multiple_of(ebase + j * K, K)
        pltpu.sync_copy(l_idx.at[pl.ds(off, K)], lidx_v)
        pltpu.sync_copy(c_idx.at[pl.ds(off, K)], cidx_v)
        d1 = pltpu.async_copy(l_tab.at[lidx_v], lrows_v, sem1)
        d2 = pltpu.async_copy(c_tab.at[cidx_v], crows_v, sem2)
        d1.wait()
        d2.wait()
        pltpu.sync_copy(lrows_v, cacc.at[cidx_v], add=True)
        pltpu.sync_copy(crows_v, lacc.at[lidx_v], add=True)
        return carry

    lax.fori_loop(0, CHUNKS, _chunk, 0)
    plsc.subcore_barrier()

    cbase = pl.multiple_of(sid * _C_STRIPE, _C_STRIPE)
    pltpu.sync_copy(cacc.at[pl.ds(cbase, _C_STRIPE)],
                    out_c.at[cid, pl.ds(cbase, _C_STRIPE)])
    lbase = pl.multiple_of(sid * _L_STRIPE, _L_STRIPE)
    pltpu.sync_copy(lacc.at[pl.ds(lbase, _L_STRIPE)],
                    out_l.at[cid, pl.ds(lbase, _L_STRIPE)])


_sc_agg = functools.partial(
    pl.kernel,
    out_type=(jax.ShapeDtypeStruct((NC, C_PAD, DIM), _f32),
              jax.ShapeDtypeStruct((NC, L_PAD, DIM), _f32)),
    mesh=plsc.VectorSubcoreMesh(core_axis_name="c", subcore_axis_name="s"),
    scratch_types=[
        pltpu.VMEM((K,), jnp.int32),
        pltpu.VMEM((K,), jnp.int32),
        pltpu.VMEM((K, DIM), _f32),
        pltpu.VMEM((K, DIM), _f32),
        pltpu.VMEM((_ZR, DIM), _f32),
        pltpu.VMEM_SHARED((C_PAD, DIM), _f32),
        pltpu.VMEM_SHARED((L_PAD, DIM), _f32),
        pltpu.SemaphoreType.DMA,
        pltpu.SemaphoreType.DMA,
    ],
)(_sc_body)


def kernel(l_size, c_size, l_edge_index, c_edge_index, l_emb, c_emb,
           l2c_W1, l2c_b1, l2c_W2, l2c_b2, c2l_W1, c2l_b1, c2l_W2, c2l_b2,
           l2l_W1, l2l_b1, l2l_W2, l2l_b2, cu_Wih, cu_Whh, cu_bih, cu_bhh,
           lu_Wih, lu_Whh, lu_bih, lu_bhh):
    pad_e = E_PAD - N_EDGES
    l_idx = jnp.concatenate(
        [l_edge_index.astype(jnp.int32),
         jnp.full((pad_e,), L_SIZE, jnp.int32)])
    c_idx = jnp.concatenate(
        [c_edge_index.astype(jnp.int32),
         jnp.full((pad_e,), C_SIZE, jnp.int32)])

    l_emb_p = jnp.pad(l_emb, ((0, L_PAD - L_SIZE), (0, 0)))
    c_emb_p = jnp.pad(c_emb, ((0, C_PAD - C_SIZE), (0, 0)))

    l_embs = [l_emb]
    c_embs = [c_emb]
    for _ in range(N_ITER):
        l_msg, l2l_msg = _pre_l(l_emb_p, l2c_W1, l2c_b1, l2c_W2, l2c_b2,
                                l2l_W1, l2l_b1, l2l_W2, l2l_b2)
        c_msg = _pre_c(c_emb_p, c2l_W1, c2l_b1, c2l_W2, c2l_b2)
        agg_c, agg_l = _sc_agg(l_idx, c_idx, l_msg, c_msg)
        c_emb_p = _gru_c(agg_c, c_emb_p, cu_Wih, cu_Whh, cu_bih, cu_bhh)
        l_emb_p = _gru_l(agg_l, l2l_msg, l_emb_p, lu_Wih, lu_Whh, lu_bih,
                         lu_bhh)
        l_embs.append(l_emb_p[:L_SIZE])
        c_embs.append(c_emb_p[:C_SIZE])

    return (jnp.stack(l_embs), jnp.stack(c_embs))


# speedup vs baseline: 2.2422x; 2.2422x over previous
import functools

import jax
import jax.numpy as jnp
from jax import lax
from jax.experimental import pallas as pl
from jax.experimental.pallas import tpu as pltpu
from jax.experimental.pallas import tpu_sc as plsc

DIM = 128
HALF = 64
L_SIZE = 10000
C_SIZE = 5000
N_EDGES = 320000
N_ITER = 4

L_PAD = 10240
C_PAD = 5120
NC = 2
NS = 16
K = 128
E_PAD = 327680
EPS = E_PAD // NS
CHUNKS = EPS // K

_f32 = jnp.float32


def _dot(x, w):
    return lax.dot_general(x, w, (((1,), (1,)), ((), ())),
                           preferred_element_type=_f32)


def _pre_l_body(x_ref, w1_ref, b1_ref, w2_ref, b2_ref,
                v1_ref, vb1_ref, v2_ref, vb2_ref, msg0_ref, msg1_ref,
                l2l_ref):
    x = x_ref[...]
    h = jnp.maximum(_dot(x, w1_ref[...]) + b1_ref[...], 0.0)
    y = _dot(h, w2_ref[...]) + b2_ref[...]
    msg0_ref[...] = y[:, :HALF]
    msg1_ref[...] = y[:, HALF:]
    xs = x.reshape(-1, 2, DIM)
    xsw = jnp.concatenate([xs[:, 1:2, :], xs[:, 0:1, :]], axis=1)
    xsw = xsw.reshape(x.shape)
    h2 = jnp.maximum(_dot(xsw, v1_ref[...]) + vb1_ref[...], 0.0)
    l2l_ref[...] = _dot(h2, v2_ref[...]) + vb2_ref[...]


def _pre_c_body(x_ref, w1_ref, b1_ref, w2_ref, b2_ref, msg0_ref, msg1_ref):
    x = x_ref[...]
    h = jnp.maximum(_dot(x, w1_ref[...]) + b1_ref[...], 0.0)
    y = _dot(h, w2_ref[...]) + b2_ref[...]
    msg0_ref[...] = y[:, :HALF]
    msg1_ref[...] = y[:, HALF:]


def _gru_gates(gi, gh, h):
    ir, iz, inn = gi[:, :DIM], gi[:, DIM:2 * DIM], gi[:, 2 * DIM:]
    hr, hz, hn = gh[:, :DIM], gh[:, DIM:2 * DIM], gh[:, 2 * DIM:]
    r = jax.nn.sigmoid(ir + hr)
    z = jax.nn.sigmoid(iz + hz)
    n = jnp.tanh(inn + r * hn)
    return (1.0 - z) * n + z * h


def _gru_c_body(agg_ref, h_ref, wih_ref, whh_ref, bih_ref, bhh_ref, out_ref):
    x = jnp.concatenate([agg_ref[0], agg_ref[1]], axis=1)
    h = h_ref[...]
    gi = _dot(x, wih_ref[...]) + bih_ref[...]
    gh = _dot(h, whh_ref[...]) + bhh_ref[...]
    out_ref[...] = _gru_gates(gi, gh, h)


def _gru_l_body(agg_ref, l2l_ref, h_ref, wih_ref, whh_ref, bih_ref, bhh_ref,
                out_ref):
    x = jnp.concatenate([agg_ref[0], agg_ref[1], l2l_ref[...]], axis=1)
    h = h_ref[...]
    gi = _dot(x, wih_ref[...]) + bih_ref[...]
    gh = _dot(h, whh_ref[...]) + bhh_ref[...]
    out_ref[...] = _gru_gates(gi, gh, h)


def _full(shape):
    return pl.BlockSpec(shape, lambda i: tuple(0 for _ in shape))


def _rows(block, width=DIM):
    return pl.BlockSpec((block, width), lambda i: (i, 0))


def _agg_spec(blk):
    return pl.BlockSpec((2, blk, HALF), lambda i: (0, i, 0))


_BLK = 512


def _pre_l(x, w1, b1, w2, b2, v1, vb1, v2, vb2):
    return pl.pallas_call(
        _pre_l_body,
        grid=(L_PAD // _BLK,),
        in_specs=[_rows(_BLK), _full((DIM, DIM)), _full((1, DIM)),
                  _full((DIM, DIM)), _full((1, DIM)), _full((DIM, DIM)),
                  _full((1, DIM)), _full((DIM, DIM)), _full((1, DIM))],
        out_specs=[_rows(_BLK, HALF), _rows(_BLK, HALF), _rows(_BLK)],
        out_shape=[jax.ShapeDtypeStruct((L_PAD, HALF), _f32),
                   jax.ShapeDtypeStruct((L_PAD, HALF), _f32),
                   jax.ShapeDtypeStruct((L_PAD, DIM), _f32)],
    )(x, w1, b1.reshape(1, DIM), w2, b2.reshape(1, DIM),
      v1, vb1.reshape(1, DIM), v2, vb2.reshape(1, DIM))


def _pre_c(x, w1, b1, w2, b2):
    return pl.pallas_call(
        _pre_c_body,
        grid=(C_PAD // _BLK,),
        in_specs=[_rows(_BLK), _full((DIM, DIM)), _full((1, DIM)),
                  _full((DIM, DIM)), _full((1, DIM))],
        out_specs=[_rows(_BLK, HALF), _rows(_BLK, HALF)],
        out_shape=[jax.ShapeDtypeStruct((C_PAD, HALF), _f32),
                   jax.ShapeDtypeStruct((C_PAD, HALF), _f32)],
    )(x, w1, b1.reshape(1, DIM), w2, b2.reshape(1, DIM))


def _gru_c(agg, h, wih, whh, bih, bhh):
    return pl.pallas_call(
        _gru_c_body,
        grid=(C_PAD // _BLK,),
        in_specs=[_agg_spec(_BLK), _rows(_BLK), _full((3 * DIM, DIM)),
                  _full((3 * DIM, DIM)), _full((1, 3 * DIM)),
                  _full((1, 3 * DIM))],
        out_specs=_rows(_BLK),
        out_shape=jax.ShapeDtypeStruct((C_PAD, DIM), _f32),
    )(agg, h, wih, whh, bih.reshape(1, 3 * DIM), bhh.reshape(1, 3 * DIM))


def _gru_l(agg, l2l, h, wih, whh, bih, bhh):
    return pl.pallas_call(
        _gru_l_body,
        grid=(L_PAD // _BLK,),
        in_specs=[_agg_spec(_BLK), _rows(_BLK), _rows(_BLK),
                  _full((3 * DIM, 2 * DIM)), _full((3 * DIM, DIM)),
                  _full((1, 3 * DIM)), _full((1, 3 * DIM))],
        out_specs=_rows(_BLK),
        out_shape=jax.ShapeDtypeStruct((L_PAD, DIM), _f32),
    )(agg, l2l, h, wih, whh, bih.reshape(1, 3 * DIM), bhh.reshape(1, 3 * DIM))


_ZR = 64
_C_STRIPE = C_PAD // NS
_L_STRIPE = L_PAD // NS


def _sc_body(l_idx, c_idx, l_tab, c_tab, out_c, out_l,
             lidx_v, cidx_v, lidx2_v, cidx2_v, lrows_v, crows_v, zbuf_v,
             cacc, lacc, sem1, sem2):
    cid = lax.axis_index("c")
    sid = lax.axis_index("s")

    zeros16 = jnp.zeros((16,), _f32)

    def _zrow(i, carry):
        for j in range(HALF // 16):
            zbuf_v[i, pl.ds(j * 16, 16)] = zeros16
        return carry

    lax.fori_loop(0, _ZR, _zrow, 0)

    def _zc(j, carry):
        base = pl.multiple_of(sid * _C_STRIPE + j * _ZR, _ZR)
        pltpu.sync_copy(zbuf_v, cacc.at[pl.ds(base, _ZR)])
        return carry

    lax.fori_loop(0, _C_STRIPE // _ZR, _zc, 0)

    def _zl(j, carry):
        base = pl.multiple_of(sid * _L_STRIPE + j * _ZR, _ZR)
        pltpu.sync_copy(zbuf_v, lacc.at[pl.ds(base, _ZR)])
        return carry

    lax.fori_loop(0, _L_STRIPE // _ZR, _zl, 0)
    plsc.subcore_barrier()

    def _chunk(j, carry):
        off = pl.multiple_of(sid * EPS + j * K, K)
        pltpu.sync_copy(l_idx.at[pl.ds(off, K)], lidx_v)
        pltpu.sync_copy(c_idx.at[pl.ds(off, K)], cidx_v)
        for g in range(K // 16):
            sl = pl.ds(g * 16, 16)
            lidx2_v[sl] = lidx_v[sl] + cid * L_PAD
            cidx2_v[sl] = cidx_v[sl] + cid * C_PAD
        d1 = pltpu.async_copy(l_tab.at[lidx2_v], lrows_v, sem1)
        d2 = pltpu.async_copy(c_tab.at[cidx2_v], crows_v, sem2)
        d1.wait()
        d2.wait()
        pltpu.sync_copy(lrows_v, cacc.at[cidx_v], add=True)
        pltpu.sync_copy(crows_v, lacc.at[lidx_v], add=True)
        return carry

    lax.fori_loop(0, CHUNKS, _chunk, 0)
    plsc.subcore_barrier()

    cbase = pl.multiple_of(sid * _C_STRIPE, _C_STRIPE)
    pltpu.sync_copy(cacc.at[pl.ds(cbase, _C_STRIPE)],
                    out_c.at[cid, pl.ds(cbase, _C_STRIPE)])
    lbase = pl.multiple_of(sid * _L_STRIPE, _L_STRIPE)
    pltpu.sync_copy(lacc.at[pl.ds(lbase, _L_STRIPE)],
                    out_l.at[cid, pl.ds(lbase, _L_STRIPE)])


_sc_agg = functools.partial(
    pl.kernel,
    out_type=(jax.ShapeDtypeStruct((NC, C_PAD, HALF), _f32),
              jax.ShapeDtypeStruct((NC, L_PAD, HALF), _f32)),
    mesh=plsc.VectorSubcoreMesh(core_axis_name="c", subcore_axis_name="s"),
    compiler_params=pltpu.CompilerParams(use_tc_tiling_on_sc=False),
    scratch_types=[
        pltpu.VMEM((K,), jnp.int32),
        pltpu.VMEM((K,), jnp.int32),
        pltpu.VMEM((K,), jnp.int32),
        pltpu.VMEM((K,), jnp.int32),
        pltpu.VMEM((K, HALF), _f32),
        pltpu.VMEM((K, HALF), _f32),
        pltpu.VMEM((_ZR, HALF), _f32),
        pltpu.VMEM_SHARED((C_PAD, HALF), _f32),
        pltpu.VMEM_SHARED((L_PAD, HALF), _f32),
        pltpu.SemaphoreType.DMA,
        pltpu.SemaphoreType.DMA,
    ],
)(_sc_body)


def kernel(l_size, c_size, l_edge_index, c_edge_index, l_emb, c_emb,
           l2c_W1, l2c_b1, l2c_W2, l2c_b2, c2l_W1, c2l_b1, c2l_W2, c2l_b2,
           l2l_W1, l2l_b1, l2l_W2, l2l_b2, cu_Wih, cu_Whh, cu_bih, cu_bhh,
           lu_Wih, lu_Whh, lu_bih, lu_bhh):
    pad_e = E_PAD - N_EDGES
    l_idx = jnp.concatenate(
        [l_edge_index.astype(jnp.int32),
         jnp.full((pad_e,), L_SIZE, jnp.int32)])
    c_idx = jnp.concatenate(
        [c_edge_index.astype(jnp.int32),
         jnp.full((pad_e,), C_SIZE, jnp.int32)])

    l_emb_p = jnp.pad(l_emb, ((0, L_PAD - L_SIZE), (0, 0)))
    c_emb_p = jnp.pad(c_emb, ((0, C_PAD - C_SIZE), (0, 0)))

    l_embs = [l_emb]
    c_embs = [c_emb]
    for _ in range(N_ITER):
        l_h0, l_h1, l2l_msg = _pre_l(l_emb_p, l2c_W1, l2c_b1, l2c_W2,
                                     l2c_b2, l2l_W1, l2l_b1, l2l_W2, l2l_b2)
        c_h0, c_h1 = _pre_c(c_emb_p, c2l_W1, c2l_b1, c2l_W2, c2l_b2)
        l_msg = jnp.concatenate([l_h0, l_h1], axis=0)
        c_msg = jnp.concatenate([c_h0, c_h1], axis=0)
        agg_c, agg_l = _sc_agg(l_idx, c_idx, l_msg, c_msg)
        c_emb_p = _gru_c(agg_c, c_emb_p, cu_Wih, cu_Whh, cu_bih, cu_bhh)
        l_emb_p = _gru_l(agg_l, l2l_msg, l_emb_p, lu_Wih, lu_Whh, lu_bih,
                         lu_bhh)
        l_embs.append(l_emb_p[:L_SIZE])
        c_embs.append(c_emb_p[:C_SIZE])

    return (jnp.stack(l_embs), jnp.stack(c_embs))


# speedup vs baseline: 3.6116x; 1.6108x over previous
import functools

import jax
import jax.numpy as jnp
from jax import lax
from jax.experimental import pallas as pl
from jax.experimental.pallas import tpu as pltpu
from jax.experimental.pallas import tpu_sc as plsc

DIM = 128
HALF = 64
L_SIZE = 10000
C_SIZE = 5000
N_EDGES = 320000
N_ITER = 4

L_PAD = 10240
C_PAD = 5120
NC = 2
NS = 16
K = 128
E_PAD = 327680
EPS = E_PAD // NS
CHUNKS = EPS // K
SUPER = 8
NSUP = CHUNKS // SUPER

_f32 = jnp.float32


def _dot(x, w):
    return lax.dot_general(x, w, (((1,), (1,)), ((), ())),
                           preferred_element_type=_f32)


def _pre_l_body(x_ref, w1_ref, b1_ref, w2_ref, b2_ref,
                v1_ref, vb1_ref, v2_ref, vb2_ref, msg0_ref, msg1_ref,
                l2l_ref):
    x = x_ref[...]
    h = jnp.maximum(_dot(x, w1_ref[...]) + b1_ref[...], 0.0)
    y = _dot(h, w2_ref[...]) + b2_ref[...]
    msg0_ref[...] = y[:, :HALF]
    msg1_ref[...] = y[:, HALF:]
    xs = x.reshape(-1, 2, DIM)
    xsw = jnp.concatenate([xs[:, 1:2, :], xs[:, 0:1, :]], axis=1)
    xsw = xsw.reshape(x.shape)
    h2 = jnp.maximum(_dot(xsw, v1_ref[...]) + vb1_ref[...], 0.0)
    l2l_ref[...] = _dot(h2, v2_ref[...]) + vb2_ref[...]


def _pre_c_body(x_ref, w1_ref, b1_ref, w2_ref, b2_ref, msg0_ref, msg1_ref):
    x = x_ref[...]
    h = jnp.maximum(_dot(x, w1_ref[...]) + b1_ref[...], 0.0)
    y = _dot(h, w2_ref[...]) + b2_ref[...]
    msg0_ref[...] = y[:, :HALF]
    msg1_ref[...] = y[:, HALF:]


def _gru_gates(gi, gh, h):
    ir, iz, inn = gi[:, :DIM], gi[:, DIM:2 * DIM], gi[:, 2 * DIM:]
    hr, hz, hn = gh[:, :DIM], gh[:, DIM:2 * DIM], gh[:, 2 * DIM:]
    r = jax.nn.sigmoid(ir + hr)
    z = jax.nn.sigmoid(iz + hz)
    n = jnp.tanh(inn + r * hn)
    return (1.0 - z) * n + z * h


def _gru_c_body(agg_ref, h_ref, wih_ref, whh_ref, bih_ref, bhh_ref, out_ref):
    x = jnp.concatenate([agg_ref[0], agg_ref[1]], axis=1)
    h = h_ref[...]
    gi = _dot(x, wih_ref[...]) + bih_ref[...]
    gh = _dot(h, whh_ref[...]) + bhh_ref[...]
    out_ref[...] = _gru_gates(gi, gh, h)


def _gru_l_body(agg_ref, l2l_ref, h_ref, wih_ref, whh_ref, bih_ref, bhh_ref,
                out_ref):
    x = jnp.concatenate([agg_ref[0], agg_ref[1], l2l_ref[...]], axis=1)
    h = h_ref[...]
    gi = _dot(x, wih_ref[...]) + bih_ref[...]
    gh = _dot(h, whh_ref[...]) + bhh_ref[...]
    out_ref[...] = _gru_gates(gi, gh, h)


def _full(shape):
    return pl.BlockSpec(shape, lambda i: tuple(0 for _ in shape))


def _rows(block, width=DIM):
    return pl.BlockSpec((block, width), lambda i: (i, 0))


def _agg_spec(blk):
    return pl.BlockSpec((2, blk, HALF), lambda i: (0, i, 0))


_BLK = 512


def _pre_l(x, w1, b1, w2, b2, v1, vb1, v2, vb2):
    return pl.pallas_call(
        _pre_l_body,
        grid=(L_PAD // _BLK,),
        in_specs=[_rows(_BLK), _full((DIM, DIM)), _full((1, DIM)),
                  _full((DIM, DIM)), _full((1, DIM)), _full((DIM, DIM)),
                  _full((1, DIM)), _full((DIM, DIM)), _full((1, DIM))],
        out_specs=[_rows(_BLK, HALF), _rows(_BLK, HALF), _rows(_BLK)],
        out_shape=[jax.ShapeDtypeStruct((L_PAD, HALF), _f32),
                   jax.ShapeDtypeStruct((L_PAD, HALF), _f32),
                   jax.ShapeDtypeStruct((L_PAD, DIM), _f32)],
    )(x, w1, b1.reshape(1, DIM), w2, b2.reshape(1, DIM),
      v1, vb1.reshape(1, DIM), v2, vb2.reshape(1, DIM))


def _pre_c(x, w1, b1, w2, b2):
    return pl.pallas_call(
        _pre_c_body,
        grid=(C_PAD // _BLK,),
        in_specs=[_rows(_BLK), _full((DIM, DIM)), _full((1, DIM)),
                  _full((DIM, DIM)), _full((1, DIM))],
        out_specs=[_rows(_BLK, HALF), _rows(_BLK, HALF)],
        out_shape=[jax.ShapeDtypeStruct((C_PAD, HALF), _f32),
                   jax.ShapeDtypeStruct((C_PAD, HALF), _f32)],
    )(x, w1, b1.reshape(1, DIM), w2, b2.reshape(1, DIM))


def _gru_c(agg, h, wih, whh, bih, bhh):
    return pl.pallas_call(
        _gru_c_body,
        grid=(C_PAD // _BLK,),
        in_specs=[_agg_spec(_BLK), _rows(_BLK), _full((3 * DIM, DIM)),
                  _full((3 * DIM, DIM)), _full((1, 3 * DIM)),
                  _full((1, 3 * DIM))],
        out_specs=_rows(_BLK),
        out_shape=jax.ShapeDtypeStruct((C_PAD, DIM), _f32),
    )(agg, h, wih, whh, bih.reshape(1, 3 * DIM), bhh.reshape(1, 3 * DIM))


def _gru_l(agg, l2l, h, wih, whh, bih, bhh):
    return pl.pallas_call(
        _gru_l_body,
        grid=(L_PAD // _BLK,),
        in_specs=[_agg_spec(_BLK), _rows(_BLK), _rows(_BLK),
                  _full((3 * DIM, 2 * DIM)), _full((3 * DIM, DIM)),
                  _full((1, 3 * DIM)), _full((1, 3 * DIM))],
        out_specs=_rows(_BLK),
        out_shape=jax.ShapeDtypeStruct((L_PAD, DIM), _f32),
    )(agg, l2l, h, wih, whh, bih.reshape(1, 3 * DIM), bhh.reshape(1, 3 * DIM))


_ZR = 64
_C_STRIPE = C_PAD // NS
_L_STRIPE = L_PAD // NS


def _sc_body(l_idx, c_idx, l_tab, c_tab, out_c, out_l,
             lidx_blk, cidx_blk, lrows_v, crows_v, zbuf_v, cacc, lacc,
             sem_a, sem_b, sem_i):
    cid = lax.axis_index("c")
    sid = lax.axis_index("s")

    zeros16 = jnp.zeros((16,), _f32)

    def _zrow(i, carry):
        for j in range(HALF // 16):
            zbuf_v[i, pl.ds(j * 16, 16)] = zeros16
        return carry

    lax.fori_loop(0, _ZR, _zrow, 0)

    def _zc(j, carry):
        base = pl.multiple_of(sid * _C_STRIPE + j * _ZR, _ZR)
        pltpu.sync_copy(zbuf_v, cacc.at[pl.ds(base, _ZR)])
        return carry

    lax.fori_loop(0, _C_STRIPE // _ZR, _zc, 0)

    def _zl(j, carry):
        base = pl.multiple_of(sid * _L_STRIPE + j * _ZR, _ZR)
        pltpu.sync_copy(zbuf_v, lacc.at[pl.ds(base, _ZR)])
        return carry

    lax.fori_loop(0, _L_STRIPE // _ZR, _zl, 0)
    plsc.subcore_barrier()

    def _idx_src(sup):
        base = pl.multiple_of(sid * CHUNKS + sup * SUPER, 8)
        return (l_idx.at[pl.ds(base, SUPER)], c_idx.at[pl.ds(base, SUPER)])

    def _fire(qq, u, buf, sem):
        d1 = pltpu.async_copy(l_tab.at[cid].at[lidx_blk.at[qq, u]],
                              lrows_v.at[buf], sem)
        d2 = pltpu.async_copy(c_tab.at[cid].at[cidx_blk.at[qq, u]],
                              crows_v.at[buf], sem)
        return d1, d2

    def _drain_scatter(qq, u, buf, sem):
        pltpu.make_async_copy(l_tab.at[cid].at[lidx_blk.at[qq, u]],
                              lrows_v.at[buf], sem).wait()
        pltpu.make_async_copy(c_tab.at[cid].at[cidx_blk.at[qq, u]],
                              crows_v.at[buf], sem).wait()
        pltpu.sync_copy(lrows_v.at[buf], cacc.at[cidx_blk.at[qq, u]],
                        add=True)
        pltpu.sync_copy(crows_v.at[buf], lacc.at[lidx_blk.at[qq, u]],
                        add=True)

    lsrc0, csrc0 = _idx_src(0)
    pltpu.sync_copy(lsrc0, lidx_blk.at[0])
    pltpu.sync_copy(csrc0, cidx_blk.at[0])
    _fire(0, 0, 0, sem_a)

    def _sup_body(s, carry):
        q = lax.rem(s, 2)
        nq = 1 - q

        @pl.when(s < NSUP - 1)
        def _():
            lsrc, csrc = _idx_src(s + 1)
            pltpu.async_copy(lsrc, lidx_blk.at[nq], sem_i)
            pltpu.async_copy(csrc, cidx_blk.at[nq], sem_i)

        for u in range(SUPER):
            p = u & 1
            if u < SUPER - 1:
                _fire(q, u + 1, 1 - p, (sem_b, sem_a)[p])
            else:
                @pl.when(s < NSUP - 1)
                def _():
                    lsrc, csrc = _idx_src(s + 1)
                    pltpu.make_async_copy(lsrc, lidx_blk.at[nq],
                                          sem_i).wait()
                    pltpu.make_async_copy(csrc, cidx_blk.at[nq],
                                          sem_i).wait()
                    _fire(nq, 0, 1 - p, (sem_b, sem_a)[p])
            _drain_scatter(q, u, p, (sem_a, sem_b)[p])
        return carry

    lax.fori_loop(0, NSUP, _sup_body, 0)
    plsc.subcore_barrier()

    cbase = pl.multiple_of(sid * _C_STRIPE, _C_STRIPE)
    pltpu.sync_copy(cacc.at[pl.ds(cbase, _C_STRIPE)],
                    out_c.at[cid, pl.ds(cbase, _C_STRIPE)])
    lbase = pl.multiple_of(sid * _L_STRIPE, _L_STRIPE)
    pltpu.sync_copy(lacc.at[pl.ds(lbase, _L_STRIPE)],
                    out_l.at[cid, pl.ds(lbase, _L_STRIPE)])


_sc_agg = functools.partial(
    pl.kernel,
    out_type=(jax.ShapeDtypeStruct((NC, C_PAD, HALF), _f32),
              jax.ShapeDtypeStruct((NC, L_PAD, HALF), _f32)),
    mesh=plsc.VectorSubcoreMesh(core_axis_name="c", subcore_axis_name="s"),
    compiler_params=pltpu.CompilerParams(use_tc_tiling_on_sc=False),
    scratch_types=[
        pltpu.VMEM((2, SUPER, K), jnp.int32),
        pltpu.VMEM((2, SUPER, K), jnp.int32),
        pltpu.VMEM((2, K, HALF), _f32),
        pltpu.VMEM((2, K, HALF), _f32),
        pltpu.VMEM((_ZR, HALF), _f32),
        pltpu.VMEM_SHARED((C_PAD, HALF), _f32),
        pltpu.VMEM_SHARED((L_PAD, HALF), _f32),
        pltpu.SemaphoreType.DMA,
        pltpu.SemaphoreType.DMA,
        pltpu.SemaphoreType.DMA,
    ],
)(_sc_body)


def kernel(l_size, c_size, l_edge_index, c_edge_index, l_emb, c_emb,
           l2c_W1, l2c_b1, l2c_W2, l2c_b2, c2l_W1, c2l_b1, c2l_W2, c2l_b2,
           l2l_W1, l2l_b1, l2l_W2, l2l_b2, cu_Wih, cu_Whh, cu_bih, cu_bhh,
           lu_Wih, lu_Whh, lu_bih, lu_bhh):
    pad_e = E_PAD - N_EDGES
    l_idx = jnp.concatenate(
        [l_edge_index.astype(jnp.int32),
         jnp.full((pad_e,), L_SIZE, jnp.int32)])
    c_idx = jnp.concatenate(
        [c_edge_index.astype(jnp.int32),
         jnp.full((pad_e,), C_SIZE, jnp.int32)])

    l_emb_p = jnp.pad(l_emb, ((0, L_PAD - L_SIZE), (0, 0)))
    c_emb_p = jnp.pad(c_emb, ((0, C_PAD - C_SIZE), (0, 0)))

    l_embs = [l_emb]
    c_embs = [c_emb]
    for _ in range(N_ITER):
        l_h0, l_h1, l2l_msg = _pre_l(l_emb_p, l2c_W1, l2c_b1, l2c_W2,
                                     l2c_b2, l2l_W1, l2l_b1, l2l_W2, l2l_b2)
        c_h0, c_h1 = _pre_c(c_emb_p, c2l_W1, c2l_b1, c2l_W2, c2l_b2)
        l_msg = jnp.stack([l_h0, l_h1], axis=0)
        c_msg = jnp.stack([c_h0, c_h1], axis=0)
        agg_c, agg_l = _sc_agg(l_idx.reshape(E_PAD // K, K),
                               c_idx.reshape(E_PAD // K, K), l_msg, c_msg)
        c_emb_p = _gru_c(agg_c, c_emb_p, cu_Wih, cu_Whh, cu_bih, cu_bhh)
        l_emb_p = _gru_l(agg_l, l2l_msg, l_emb_p, lu_Wih, lu_Whh, lu_bih,
                         lu_bhh)
        l_embs.append(l_emb_p[:L_SIZE])
        c_embs.append(c_emb_p[:C_SIZE])

    return (jnp.stack(l_embs), jnp.stack(c_embs))
